# Initial kernel scaffold; baseline (speedup 1.0000x reference)
import jax
import jax.numpy as jnp
from jax.experimental import pallas as pl


def kernel(user_ids, item_ids, user_table, item_table, W1, b1, W2, b2, W3, b3):
    raise NotImplementedError("write your pallas kernel here")


# speedup vs baseline: 4.3330x; 4.3330x over previous
import functools

import jax
import jax.numpy as jnp
from jax import lax
from jax.experimental import pallas as pl
from jax.experimental.pallas import tpu as pltpu
from jax.experimental.pallas import tpu_sc as plsc

EMB = 128
CHUNK = 128


def _gather_body(uid_hbm, iid_hbm, utab_hbm, itab_hbm, out_u, out_i,
                 idx_v, rows_v, sem, *, rows_per_worker, num_cores):
    wid = lax.axis_index("s") * num_cores + lax.axis_index("c")
    base = wid * rows_per_worker
    n_chunks = rows_per_worker // CHUNK
    for ids_hbm, tab_hbm, out_hbm in ((uid_hbm, utab_hbm, out_u),
                                      (iid_hbm, itab_hbm, out_i)):
        for j in range(n_chunks):
            off = base + j * CHUNK
            pltpu.sync_copy(ids_hbm.at[pl.ds(off, CHUNK)], idx_v)
            pltpu.async_copy(tab_hbm.at[idx_v], rows_v, sem).wait()
            pltpu.sync_copy(rows_v, out_hbm.at[pl.ds(off, CHUNK)])


def _sc_gather(user_ids, item_ids, user_table, item_table):
    batch = user_ids.shape[0]
    info = plsc.get_sparse_core_info()
    nw = info.num_cores * info.num_subcores
    rows_per_worker = batch // nw
    mesh = plsc.VectorSubcoreMesh(core_axis_name="c", subcore_axis_name="s")
    out = jax.ShapeDtypeStruct((batch, EMB), jnp.float32)
    body = functools.partial(_gather_body, rows_per_worker=rows_per_worker,
                             num_cores=info.num_cores)
    return pl.kernel(
        body,
        out_type=(out, out),
        mesh=mesh,
        scratch_types=[
            pltpu.VMEM((CHUNK,), jnp.int32),
            pltpu.VMEM((CHUNK, EMB), jnp.float32),
            pltpu.SemaphoreType.DMA,
        ],
    )(user_ids, item_ids, user_table, item_table)


def _mlp_body(u_ref, i_ref, w1a_ref, w1b_ref, b1_ref, w2_ref, b2_ref,
              w3_ref, b3_ref, out_ref):
    h = jnp.dot(u_ref[...], w1a_ref[...], preferred_element_type=jnp.float32)
    h += jnp.dot(i_ref[...], w1b_ref[...], preferred_element_type=jnp.float32)
    h = jnp.maximum(h + b1_ref[...], 0.0)
    h = jnp.dot(h, w2_ref[...], preferred_element_type=jnp.float32)
    h = jnp.maximum(h + b2_ref[...], 0.0)
    logit = jnp.sum(h * w3_ref[...], axis=1) + b3_ref[0, 0]
    out_ref[...] = jax.nn.sigmoid(logit)


def _tc_mlp(u, i, W1, b1, W2, b2, W3, b3):
    batch = u.shape[0]
    block = 2048
    grid = (batch // block,)
    w1a = W1[:EMB]
    w1b = W1[EMB:]
    b1r = b1.reshape(1, -1)
    b2r = b2.reshape(1, -1)
    w3r = W3.reshape(1, -1)
    b3r = b3.reshape(1, 1)
    full = lambda shape: pl.BlockSpec(shape, lambda k: (0,) * len(shape))
    return pl.pallas_call(
        _mlp_body,
        grid=grid,
        in_specs=[
            pl.BlockSpec((block, EMB), lambda k: (k, 0)),
            pl.BlockSpec((block, EMB), lambda k: (k, 0)),
            full(w1a.shape),
            full(w1b.shape),
            full(b1r.shape),
            full(W2.shape),
            full(b2r.shape),
            full(w3r.shape),
            full(b3r.shape),
        ],
        out_specs=pl.BlockSpec((block,), lambda k: (k,)),
        out_shape=jax.ShapeDtypeStruct((batch,), jnp.float32),
    )(u, i, w1a, w1b, b1r, W2, b2r, w3r, b3r)


def kernel(user_ids, item_ids, user_table, item_table, W1, b1, W2, b2, W3, b3):
    u, i = _sc_gather(user_ids.astype(jnp.int32), item_ids.astype(jnp.int32),
                      user_table, item_table)
    return _tc_mlp(u, i, W1, b1, W2, b2, W3, b3)
